# baseline (speedup 1.0000x reference)
import jax
import jax.numpy as jnp
from jax import lax
from jax.experimental import pallas as pl
from jax.experimental.pallas import tpu as pltpu
from jax.experimental.pallas import tpu_sc as plsc

B = 16384
D = 192
DP = 256
H = 768
NC = 2
NS = 16
NW = NC * NS
BPW = B // NW
CH = 256


def _gather_body(cat_i, brand_i, item_i, seller_i,
                 t_cat, t_brand, t_item, t_seller,
                 o_c, o_b, o_i, o_s,
                 idx_v, buf, sem):
    wid = lax.axis_index("s") * NC + lax.axis_index("c")
    base = wid * BPW
    for idx_hbm, tab, out in ((cat_i, t_cat, o_c), (brand_i, t_brand, o_b),
                              (item_i, t_item, o_i), (seller_i, t_seller, o_s)):
        pltpu.sync_copy(idx_hbm.at[pl.ds(base, BPW)], idx_v)
        for ch in range(BPW // CH):
            lo = ch * CH
            ids = idx_v.at[pl.ds(lo, CH)]
            pltpu.async_copy(tab.at[ids], buf, sem).wait()
            pltpu.sync_copy(buf, out.at[pl.ds(base + lo, CH)])


_gather = pl.kernel(
    _gather_body,
    mesh=plsc.VectorSubcoreMesh(core_axis_name="c", subcore_axis_name="s"),
    out_type=[jax.ShapeDtypeStruct((B, DP), jnp.float32)] * 4,
    scratch_types=[
        pltpu.VMEM((BPW,), jnp.int32),
        pltpu.VMEM((CH, DP), jnp.float32),
        pltpu.SemaphoreType.DMA,
    ],
)


BM = 1024


def _mlp_body(xc, xb, xi, xs, w1, b1, w2, b2, out):
    x = jnp.concatenate([xc[...], xb[...], xi[...], xs[...]], axis=-1)
    h = jnp.maximum(
        jnp.dot(x, w1[...], preferred_element_type=jnp.float32) + b1[...], 0.0)
    out[...] = jnp.dot(h, w2[...], preferred_element_type=jnp.float32) + b2[...]


_mlp = pl.pallas_call(
    _mlp_body,
    grid=(B // BM,),
    in_specs=[pl.BlockSpec((BM, DP), lambda i: (i, 0)) for _ in range(4)] + [
        pl.BlockSpec((4 * DP, H), lambda i: (0, 0)),
        pl.BlockSpec((1, H), lambda i: (0, 0)),
        pl.BlockSpec((H, H), lambda i: (0, 0)),
        pl.BlockSpec((1, H), lambda i: (0, 0)),
    ],
    out_specs=pl.BlockSpec((BM, H), lambda i: (i, 0)),
    out_shape=jax.ShapeDtypeStruct((B, H), jnp.float32),
)


def kernel(category, brand, item_id, seller,
           emb_category, emb_brand, emb_item_id, emb_seller,
           W1, b1, W2, b2):
    idx = [x.astype(jnp.int32) for x in (category, brand, item_id, seller)]
    tabs = [jnp.pad(e, ((0, 0), (0, DP - D))) for e in
            (emb_category, emb_brand, emb_item_id, emb_seller)]
    parts = _gather(*idx, *tabs)
    w1z = jnp.pad(W1.reshape(4, D, H), ((0, 0), (0, DP - D), (0, 0)))
    w1z = w1z.reshape(4 * DP, H)
    return _mlp(*parts, w1z, b1.reshape(1, H), W2, b2.reshape(1, H))


# speedup vs baseline: 4.0559x; 4.0559x over previous
import jax
import jax.numpy as jnp
from jax import lax
from jax.experimental import pallas as pl
from jax.experimental.pallas import tpu as pltpu
from jax.experimental.pallas import tpu_sc as plsc

B = 16384
D = 192
DP = 256
H = 768
NC = 2
NS = 16
NW = NC * NS
BPW = B // NW
CH = 256


def _gather_body(cat_i, brand_i, item_i, seller_i,
                 t_cat, t_brand, t_item, t_seller,
                 o_c, o_b, o_i, o_s,
                 idx_v, buf, sem):
    wid = lax.axis_index("s") * NC + lax.axis_index("c")
    base = wid * BPW
    for idx_hbm, tab, out in ((cat_i, t_cat, o_c), (brand_i, t_brand, o_b),
                              (item_i, t_item, o_i), (seller_i, t_seller, o_s)):
        pltpu.sync_copy(idx_hbm.at[pl.ds(base, BPW)], idx_v)
        for ch in range(BPW // CH):
            lo = ch * CH
            ids = idx_v.at[pl.ds(lo, CH)]
            pltpu.async_copy(tab.at[ids], buf, sem).wait()
            pltpu.sync_copy(buf, out.at[pl.ds(base + lo, CH)])


_gather = pl.kernel(
    _gather_body,
    mesh=plsc.VectorSubcoreMesh(core_axis_name="c", subcore_axis_name="s"),
    out_type=[jax.ShapeDtypeStruct((B, DP), jnp.float32)] * 4,
    scratch_types=[
        pltpu.VMEM((BPW,), jnp.int32),
        pltpu.VMEM((CH, DP), jnp.float32),
        pltpu.SemaphoreType.DMA,
    ],
)


BM = 1024


def _mlp_body(xc, xb, xi, xs, w1, b1, w2, b2, out):
    x = jnp.concatenate([xc[...], xb[...], xi[...], xs[...]], axis=-1)
    h = jnp.maximum(
        jnp.dot(x, w1[...], preferred_element_type=jnp.float32) + b1[...], 0.0)
    out[...] = jnp.dot(h, w2[...], preferred_element_type=jnp.float32) + b2[...]


_mlp = pl.pallas_call(
    _mlp_body,
    grid=(B // BM,),
    in_specs=[pl.BlockSpec((BM, DP), lambda i: (i, 0)) for _ in range(4)] + [
        pl.BlockSpec((4 * DP, H), lambda i: (0, 0)),
        pl.BlockSpec((1, H), lambda i: (0, 0)),
        pl.BlockSpec((H, H), lambda i: (0, 0)),
        pl.BlockSpec((1, H), lambda i: (0, 0)),
    ],
    out_specs=pl.BlockSpec((BM, H), lambda i: (i, 0)),
    out_shape=jax.ShapeDtypeStruct((B, H), jnp.float32),
)


BV = 2048


def _relayout_body(xt, out):
    t = xt[...].T
    out[...] = jnp.concatenate(
        [t, jnp.zeros((t.shape[0], DP - D), jnp.float32)], axis=-1)


def _relayout(emb):
    v = emb.shape[0]
    nb = (v + BV - 1) // BV
    return pl.pallas_call(
        _relayout_body,
        grid=(nb,),
        in_specs=[pl.BlockSpec((D, BV), lambda i: (0, i))],
        out_specs=pl.BlockSpec((BV, DP), lambda i: (i, 0)),
        out_shape=jax.ShapeDtypeStruct((v, DP), jnp.float32),
    )(emb.T)


def kernel(category, brand, item_id, seller,
           emb_category, emb_brand, emb_item_id, emb_seller,
           W1, b1, W2, b2):
    idx = [x.astype(jnp.int32) for x in (category, brand, item_id, seller)]
    tabs = [_relayout(e) for e in
            (emb_category, emb_brand, emb_item_id, emb_seller)]
    parts = _gather(*idx, *tabs)
    w1z = jnp.pad(W1.reshape(4, D, H), ((0, 0), (0, DP - D), (0, 0)))
    w1z = w1z.reshape(4 * DP, H)
    return _mlp(*parts, w1z, b1.reshape(1, H), W2, b2.reshape(1, H))


# speedup vs baseline: 4.1227x; 1.0165x over previous
import jax
import jax.numpy as jnp
from jax import lax
from jax.experimental import pallas as pl
from jax.experimental.pallas import tpu as pltpu
from jax.experimental.pallas import tpu_sc as plsc

B = 16384
D = 192
DP = 256
H = 768
NC = 2
NS = 16
NW = NC * NS
BPW = B // NW
CH = 128


NCH = BPW // CH


def _gather_body(idx_hbm, tab, out, idx_v, buf0, buf1, sem0, sem1):
    wid = lax.axis_index("s") * NC + lax.axis_index("c")
    base = wid * BPW
    pltpu.sync_copy(idx_hbm.at[pl.ds(base, BPW)], idx_v)
    bufs, sems = (buf0, buf1), (sem0, sem1)
    copies = [None] * NCH
    for ch in range(NCH):
        ids = idx_v.at[pl.ds(ch * CH, CH)]
        copies[ch] = pltpu.async_copy(tab.at[ids], bufs[ch % 2], sems[ch % 2])
        if ch > 0:
            copies[ch - 1].wait()
            pltpu.sync_copy(bufs[(ch - 1) % 2],
                            out.at[pl.ds(base + (ch - 1) * CH, CH)])
    copies[NCH - 1].wait()
    pltpu.sync_copy(bufs[(NCH - 1) % 2],
                    out.at[pl.ds(base + (NCH - 1) * CH, CH)])


_gather1 = pl.kernel(
    _gather_body,
    mesh=plsc.VectorSubcoreMesh(core_axis_name="c", subcore_axis_name="s"),
    out_type=jax.ShapeDtypeStruct((B, DP), jnp.float32),
    scratch_types=[
        pltpu.VMEM((BPW,), jnp.int32),
        pltpu.VMEM((CH, DP), jnp.float32),
        pltpu.VMEM((CH, DP), jnp.float32),
        pltpu.SemaphoreType.DMA,
        pltpu.SemaphoreType.DMA,
    ],
)


BM = 1024


def _mlp_body(xc, xb, xi, xs, w1, b1, w2, b2, out):
    x = jnp.concatenate([xc[...], xb[...], xi[...], xs[...]], axis=-1)
    h = jnp.maximum(
        jnp.dot(x, w1[...], preferred_element_type=jnp.float32) + b1[...], 0.0)
    out[...] = jnp.dot(h, w2[...], preferred_element_type=jnp.float32) + b2[...]


_mlp = pl.pallas_call(
    _mlp_body,
    grid=(B // BM,),
    in_specs=[pl.BlockSpec((BM, DP), lambda i: (i, 0)) for _ in range(4)] + [
        pl.BlockSpec((4 * DP, H), lambda i: (0, 0)),
        pl.BlockSpec((1, H), lambda i: (0, 0)),
        pl.BlockSpec((H, H), lambda i: (0, 0)),
        pl.BlockSpec((1, H), lambda i: (0, 0)),
    ],
    out_specs=pl.BlockSpec((BM, H), lambda i: (i, 0)),
    out_shape=jax.ShapeDtypeStruct((B, H), jnp.float32),
)


BV = 2048


def _relayout_body(xt, out):
    t = xt[...].T
    out[...] = jnp.concatenate(
        [t, jnp.zeros((t.shape[0], DP - D), jnp.float32)], axis=-1)


def _relayout(emb):
    v = emb.shape[0]
    nb = (v + BV - 1) // BV
    return pl.pallas_call(
        _relayout_body,
        grid=(nb,),
        in_specs=[pl.BlockSpec((D, BV), lambda i: (0, i))],
        out_specs=pl.BlockSpec((BV, DP), lambda i: (i, 0)),
        out_shape=jax.ShapeDtypeStruct((v, DP), jnp.float32),
    )(emb.T)


def kernel(category, brand, item_id, seller,
           emb_category, emb_brand, emb_item_id, emb_seller,
           W1, b1, W2, b2):
    idx = [x.astype(jnp.int32) for x in (category, brand, item_id, seller)]
    parts = []
    for i, e in enumerate((emb_category, emb_brand, emb_item_id, emb_seller)):
        parts.append(_gather1(idx[i], _relayout(e)))
    w1z = jnp.pad(W1.reshape(4, D, H), ((0, 0), (0, DP - D), (0, 0)))
    w1z = w1z.reshape(4 * DP, H)
    return _mlp(*parts, w1z, b1.reshape(1, H), W2, b2.reshape(1, H))


# speedup vs baseline: 5.0715x; 1.2302x over previous
import jax
import jax.numpy as jnp
from jax import lax
from jax.experimental import pallas as pl
from jax.experimental.pallas import tpu as pltpu
from jax.experimental.pallas import tpu_sc as plsc

B = 16384
D = 192
DP = 256
H = 768
NC = 2
NS = 16
NW = NC * NS
BPW = B // NW
CH = 256


NCH = BPW // CH


def _gather_body(idx_hbm, tab, out, idx_v, buf0, buf1, sem0, sem1):
    wid = lax.axis_index("s") * NC + lax.axis_index("c")
    base = wid * BPW
    pltpu.sync_copy(idx_hbm.at[pl.ds(base, BPW)], idx_v)
    bufs, sems = (buf0, buf1), (sem0, sem1)
    copies = [None] * NCH
    for ch in range(NCH):
        ids = idx_v.at[pl.ds(ch * CH, CH)]
        copies[ch] = pltpu.async_copy(tab.at[ids], bufs[ch % 2], sems[ch % 2])
        if ch > 0:
            copies[ch - 1].wait()
            pltpu.sync_copy(bufs[(ch - 1) % 2],
                            out.at[pl.ds(base + (ch - 1) * CH, CH)])
    copies[NCH - 1].wait()
    pltpu.sync_copy(bufs[(NCH - 1) % 2],
                    out.at[pl.ds(base + (NCH - 1) * CH, CH)])


_gather1 = pl.kernel(
    _gather_body,
    mesh=plsc.VectorSubcoreMesh(core_axis_name="c", subcore_axis_name="s"),
    out_type=jax.ShapeDtypeStruct((B, DP // 2), jnp.float32),
    scratch_types=[
        pltpu.VMEM((BPW,), jnp.int32),
        pltpu.VMEM((CH, DP // 2), jnp.float32),
        pltpu.VMEM((CH, DP // 2), jnp.float32),
        pltpu.SemaphoreType.DMA,
        pltpu.SemaphoreType.DMA,
    ],
)


BM = 1024


def _unpack2(p):
    w = jax.lax.bitcast_convert_type(p[...], jnp.int32)
    lo = jax.lax.bitcast_convert_type(
        jax.lax.shift_left(w, 16), jnp.float32)
    hi = jax.lax.bitcast_convert_type(w & jnp.int32(-65536), jnp.float32)
    return lo, hi


def _mlp_body(xc, xb, xi, xs, w1, b1, w2, b2, out):
    pieces = []
    for p in (xc, xb, xi, xs):
        lo, hi = _unpack2(p)
        pieces.append(lo)
        pieces.append(hi)
    x = jnp.concatenate(pieces, axis=-1).astype(jnp.bfloat16)
    h = jnp.maximum(
        jnp.dot(x, w1[...], preferred_element_type=jnp.float32) + b1[...], 0.0)
    out[...] = jnp.dot(h, w2[...], preferred_element_type=jnp.float32) + b2[...]


_mlp = pl.pallas_call(
    _mlp_body,
    grid=(B // BM,),
    in_specs=[pl.BlockSpec((BM, DP // 2), lambda i: (i, 0)) for _ in range(4)] + [
        pl.BlockSpec((4 * DP, H), lambda i: (0, 0)),
        pl.BlockSpec((1, H), lambda i: (0, 0)),
        pl.BlockSpec((H, H), lambda i: (0, 0)),
        pl.BlockSpec((1, H), lambda i: (0, 0)),
    ],
    out_specs=pl.BlockSpec((BM, H), lambda i: (i, 0)),
    out_shape=jax.ShapeDtypeStruct((B, H), jnp.float32),
)


BV = 2048


def _pack_bits(x):
    r = x.astype(jnp.bfloat16).astype(jnp.float32)
    return jax.lax.bitcast_convert_type(r, jnp.int32)


def _relayout_body(xt, out):
    t = xt[...].T
    lo = _pack_bits(t[:, :128])
    hi = _pack_bits(jnp.concatenate(
        [t[:, 128:], jnp.zeros((t.shape[0], DP - D), jnp.float32)], axis=-1))
    word = jax.lax.shift_right_logical(lo, 16) | (hi & jnp.int32(-65536))
    out[...] = jax.lax.bitcast_convert_type(word, jnp.float32)


def _relayout(emb):
    v = emb.shape[0]
    nb = (v + BV - 1) // BV
    return pl.pallas_call(
        _relayout_body,
        grid=(nb,),
        in_specs=[pl.BlockSpec((D, BV), lambda i: (0, i))],
        out_specs=pl.BlockSpec((BV, DP // 2), lambda i: (i, 0)),
        out_shape=jax.ShapeDtypeStruct((v, DP // 2), jnp.float32),
    )(emb.T)


def kernel(category, brand, item_id, seller,
           emb_category, emb_brand, emb_item_id, emb_seller,
           W1, b1, W2, b2):
    idx = [x.astype(jnp.int32) for x in (category, brand, item_id, seller)]
    parts = []
    for i, e in enumerate((emb_category, emb_brand, emb_item_id, emb_seller)):
        parts.append(_gather1(idx[i], _relayout(e)))
    w1z = jnp.pad(W1.reshape(4, D, H), ((0, 0), (0, DP - D), (0, 0)))
    w1z = w1z.reshape(4 * DP, H).astype(jnp.bfloat16)
    return _mlp(*parts, w1z, b1.reshape(1, H), W2, b2.reshape(1, H))


# speedup vs baseline: 6.0697x; 1.1968x over previous
import jax
import jax.numpy as jnp
from jax import lax
from jax.experimental import pallas as pl
from jax.experimental.pallas import tpu as pltpu
from jax.experimental.pallas import tpu_sc as plsc

B = 16384
D = 192
DP = 256
H = 768
NC = 2
NS = 16
NW = NC * NS
BPW = B // NW
CH = 256


NCH = BPW // CH


def _gather_body(idx_hbm, tab, out, idx_v, buf0, buf1, sem0, sem1):
    wid = lax.axis_index("s") * NC + lax.axis_index("c")
    base = wid * BPW
    pltpu.sync_copy(idx_hbm.at[pl.ds(base, BPW)], idx_v)
    bufs, sems = (buf0, buf1), (sem0, sem1)
    copies = [None] * NCH
    for ch in range(NCH):
        ids = idx_v.at[pl.ds(ch * CH, CH)]
        copies[ch] = pltpu.async_copy(tab.at[ids], bufs[ch % 2], sems[ch % 2])
        if ch > 0:
            copies[ch - 1].wait()
            pltpu.sync_copy(bufs[(ch - 1) % 2],
                            out.at[pl.ds(base + (ch - 1) * CH, CH)])
    copies[NCH - 1].wait()
    pltpu.sync_copy(bufs[(NCH - 1) % 2],
                    out.at[pl.ds(base + (NCH - 1) * CH, CH)])


_gather1 = pl.kernel(
    _gather_body,
    mesh=plsc.VectorSubcoreMesh(core_axis_name="c", subcore_axis_name="s"),
    out_type=jax.ShapeDtypeStruct((B, DP // 2), jnp.float32),
    scratch_types=[
        pltpu.VMEM((BPW,), jnp.int32),
        pltpu.VMEM((CH, DP // 2), jnp.float32),
        pltpu.VMEM((CH, DP // 2), jnp.float32),
        pltpu.SemaphoreType.DMA,
        pltpu.SemaphoreType.DMA,
    ],
)


BM = 1024


def _unpack2(p):
    w = jax.lax.bitcast_convert_type(p[...], jnp.int32)
    lo = jax.lax.bitcast_convert_type(
        jax.lax.shift_left(w, 16), jnp.float32)
    hi = jax.lax.bitcast_convert_type(w & jnp.int32(-65536), jnp.float32)
    return lo, hi


def _mlp_body(xc, xb, xi, xs, w1, b1, w2, b2, out):
    pieces = []
    for p in (xc, xb, xi, xs):
        lo, hi = _unpack2(p)
        pieces.append(lo)
        pieces.append(hi)
    x = jnp.concatenate(pieces, axis=-1).astype(jnp.bfloat16)
    h = jnp.maximum(
        jnp.dot(x, w1[...], preferred_element_type=jnp.float32) + b1[...],
        0.0).astype(jnp.bfloat16)
    out[...] = jnp.dot(h, w2[...], preferred_element_type=jnp.float32) + b2[...]


_mlp = pl.pallas_call(
    _mlp_body,
    grid=(B // BM,),
    in_specs=[pl.BlockSpec((BM, DP // 2), lambda i: (i, 0)) for _ in range(4)] + [
        pl.BlockSpec((4 * DP, H), lambda i: (0, 0)),
        pl.BlockSpec((1, H), lambda i: (0, 0)),
        pl.BlockSpec((H, H), lambda i: (0, 0)),
        pl.BlockSpec((1, H), lambda i: (0, 0)),
    ],
    out_specs=pl.BlockSpec((BM, H), lambda i: (i, 0)),
    out_shape=jax.ShapeDtypeStruct((B, H), jnp.float32),
)


BV = 4096


def _pack_bits(x):
    r = x.astype(jnp.bfloat16).astype(jnp.float32)
    return jax.lax.bitcast_convert_type(r, jnp.int32)


def _relayout_body(xt, out):
    t = xt[...].T
    lo = _pack_bits(t[:, :128])
    hi = _pack_bits(jnp.concatenate(
        [t[:, 128:], jnp.zeros((t.shape[0], DP - D), jnp.float32)], axis=-1))
    word = jax.lax.shift_right_logical(lo, 16) | (hi & jnp.int32(-65536))
    out[...] = jax.lax.bitcast_convert_type(word, jnp.float32)


def _relayout(emb):
    v = emb.shape[0]
    nb = (v + BV - 1) // BV
    return pl.pallas_call(
        _relayout_body,
        grid=(nb,),
        in_specs=[pl.BlockSpec((D, BV), lambda i: (0, i))],
        out_specs=pl.BlockSpec((BV, DP // 2), lambda i: (i, 0)),
        out_shape=jax.ShapeDtypeStruct((v, DP // 2), jnp.float32),
    )(emb.T)


def kernel(category, brand, item_id, seller,
           emb_category, emb_brand, emb_item_id, emb_seller,
           W1, b1, W2, b2):
    idx = [x.astype(jnp.int32) for x in (category, brand, item_id, seller)]
    parts = []
    prev = None
    for i, e in enumerate((emb_category, emb_brand, emb_item_id, emb_seller)):
        if prev is not None:
            e, _ = jax.lax.optimization_barrier((e, prev))
        t = _relayout(e)
        prev = t
        parts.append(_gather1(idx[i], t))
    w1z = jnp.pad(W1.reshape(4, D, H), ((0, 0), (0, DP - D), (0, 0)))
    w1z = w1z.reshape(4 * DP, H).astype(jnp.bfloat16)
    return _mlp(*parts, w1z, b1.reshape(1, H),
                W2.astype(jnp.bfloat16), b2.reshape(1, H))


# speedup vs baseline: 6.3645x; 1.0486x over previous
import jax
import jax.numpy as jnp
from jax import lax
from jax.experimental import pallas as pl
from jax.experimental.pallas import tpu as pltpu
from jax.experimental.pallas import tpu_sc as plsc

B = 16384
D = 192
DP = 256
H = 768
NC = 2
NS = 16
NW = NC * NS
BPW = B // NW
CH = 256


NCH = BPW // CH


def _gather_body(idx_hbm, tab, out, idx_v, buf0, buf1, sem0, sem1):
    wid = lax.axis_index("s") * NC + lax.axis_index("c")
    base = wid * BPW
    pltpu.sync_copy(idx_hbm.at[pl.ds(base, BPW)], idx_v)
    bufs, sems = (buf0, buf1), (sem0, sem1)
    copies = [None] * NCH
    for ch in range(NCH):
        ids = idx_v.at[pl.ds(ch * CH, CH)]
        copies[ch] = pltpu.async_copy(tab.at[ids], bufs[ch % 2], sems[ch % 2])
        if ch > 0:
            copies[ch - 1].wait()
            pltpu.sync_copy(bufs[(ch - 1) % 2],
                            out.at[pl.ds(base + (ch - 1) * CH, CH)])
    copies[NCH - 1].wait()
    pltpu.sync_copy(bufs[(NCH - 1) % 2],
                    out.at[pl.ds(base + (NCH - 1) * CH, CH)])


_gather1 = pl.kernel(
    _gather_body,
    mesh=plsc.VectorSubcoreMesh(core_axis_name="c", subcore_axis_name="s"),
    out_type=jax.ShapeDtypeStruct((B, DP // 2), jnp.float32),
    scratch_types=[
        pltpu.VMEM((BPW,), jnp.int32),
        pltpu.VMEM((CH, DP // 2), jnp.float32),
        pltpu.VMEM((CH, DP // 2), jnp.float32),
        pltpu.SemaphoreType.DMA,
        pltpu.SemaphoreType.DMA,
    ],
)


BM = 2048


def _unpack2(p):
    w = jax.lax.bitcast_convert_type(p[...], jnp.int32)
    lo = jax.lax.bitcast_convert_type(
        jax.lax.shift_left(w, 16), jnp.float32)
    hi = jax.lax.bitcast_convert_type(w & jnp.int32(-65536), jnp.float32)
    return lo, hi


def _mlp_body(xc, xb, xi, xs, w1, b1, w2, b2, out):
    pieces = []
    for p in (xc, xb, xi, xs):
        lo, hi = _unpack2(p)
        pieces.append(lo)
        pieces.append(hi)
    x = jnp.concatenate(pieces, axis=-1).astype(jnp.bfloat16)
    h = jnp.maximum(
        jnp.dot(x, w1[...], preferred_element_type=jnp.float32) + b1[...],
        0.0).astype(jnp.bfloat16)
    out[...] = jnp.dot(h, w2[...], preferred_element_type=jnp.float32) + b2[...]


_mlp = pl.pallas_call(
    _mlp_body,
    grid=(B // BM,),
    in_specs=[pl.BlockSpec((BM, DP // 2), lambda i: (i, 0)) for _ in range(4)] + [
        pl.BlockSpec((4 * DP, H), lambda i: (0, 0)),
        pl.BlockSpec((1, H), lambda i: (0, 0)),
        pl.BlockSpec((H, H), lambda i: (0, 0)),
        pl.BlockSpec((1, H), lambda i: (0, 0)),
    ],
    out_specs=pl.BlockSpec((BM, H), lambda i: (i, 0)),
    out_shape=jax.ShapeDtypeStruct((B, H), jnp.float32),
)


BV = 8192


def _pack_bits(x):
    r = x.astype(jnp.bfloat16).astype(jnp.float32)
    return jax.lax.bitcast_convert_type(r, jnp.int32)


def _relayout_body(xt, out):
    t = xt[...].T
    lo = _pack_bits(t[:, :128])
    hi = _pack_bits(jnp.concatenate(
        [t[:, 128:], jnp.zeros((t.shape[0], DP - D), jnp.float32)], axis=-1))
    word = jax.lax.shift_right_logical(lo, 16) | (hi & jnp.int32(-65536))
    out[...] = jax.lax.bitcast_convert_type(word, jnp.float32)


def _relayout(emb):
    v = emb.shape[0]
    nb = (v + BV - 1) // BV
    return pl.pallas_call(
        _relayout_body,
        grid=(nb,),
        in_specs=[pl.BlockSpec((D, BV), lambda i: (0, i))],
        out_specs=pl.BlockSpec((BV, DP // 2), lambda i: (i, 0)),
        out_shape=jax.ShapeDtypeStruct((v, DP // 2), jnp.float32),
    )(emb.T)


def kernel(category, brand, item_id, seller,
           emb_category, emb_brand, emb_item_id, emb_seller,
           W1, b1, W2, b2):
    idx = [x.astype(jnp.int32) for x in (category, brand, item_id, seller)]
    parts = []
    prev = None
    for i, e in enumerate((emb_category, emb_brand, emb_item_id, emb_seller)):
        if prev is not None:
            e, _ = jax.lax.optimization_barrier((e, prev))
        t = _relayout(e)
        prev = t
        parts.append(_gather1(idx[i], t))
    w1z = jnp.pad(W1.reshape(4, D, H), ((0, 0), (0, DP - D), (0, 0)))
    w1z = w1z.reshape(4 * DP, H).astype(jnp.bfloat16)
    return _mlp(*parts, w1z, b1.reshape(1, H),
                W2.astype(jnp.bfloat16), b2.reshape(1, H))


# speedup vs baseline: 6.5199x; 1.0244x over previous
import jax
import jax.numpy as jnp
from jax import lax
from jax.experimental import pallas as pl
from jax.experimental.pallas import tpu as pltpu
from jax.experimental.pallas import tpu_sc as plsc

B = 16384
D = 192
DP = 256
H = 768
NC = 2
NS = 16
NW = NC * NS
BPW = B // NW
CH = 256


NCH = BPW // CH


def _gather_body(idx_hbm, tab, out, idx_v, buf0, buf1, sem0, sem1):
    wid = lax.axis_index("s") * NC + lax.axis_index("c")
    base = wid * BPW
    pltpu.sync_copy(idx_hbm.at[pl.ds(base, BPW)], idx_v)
    bufs, sems = (buf0, buf1), (sem0, sem1)
    copies = [None] * NCH
    for ch in range(NCH):
        ids = idx_v.at[pl.ds(ch * CH, CH)]
        copies[ch] = pltpu.async_copy(tab.at[ids], bufs[ch % 2], sems[ch % 2])
        if ch > 0:
            copies[ch - 1].wait()
            pltpu.sync_copy(bufs[(ch - 1) % 2],
                            out.at[pl.ds(base + (ch - 1) * CH, CH)])
    copies[NCH - 1].wait()
    pltpu.sync_copy(bufs[(NCH - 1) % 2],
                    out.at[pl.ds(base + (NCH - 1) * CH, CH)])


_gather1 = pl.kernel(
    _gather_body,
    mesh=plsc.VectorSubcoreMesh(core_axis_name="c", subcore_axis_name="s"),
    out_type=jax.ShapeDtypeStruct((B, DP // 2), jnp.float32),
    scratch_types=[
        pltpu.VMEM((BPW,), jnp.int32),
        pltpu.VMEM((CH, DP // 2), jnp.float32),
        pltpu.VMEM((CH, DP // 2), jnp.float32),
        pltpu.SemaphoreType.DMA,
        pltpu.SemaphoreType.DMA,
    ],
)


BM = 2048


def _unpack2(p):
    w = jax.lax.bitcast_convert_type(p[...], jnp.int32)
    lo = jax.lax.bitcast_convert_type(
        jax.lax.shift_left(w, 16), jnp.float32)
    hi = jax.lax.bitcast_convert_type(w & jnp.int32(-65536), jnp.float32)
    return lo, hi


def _mlp_body(xc, xb, xi, xs, w1, b1, w2, b2, out):
    pieces = []
    for p in (xc, xb, xi, xs):
        lo, hi = _unpack2(p)
        pieces.append(lo)
        pieces.append(hi[:, :D - 128])
    x = jnp.concatenate(pieces, axis=-1).astype(jnp.bfloat16)
    h = jnp.maximum(
        jnp.dot(x, w1[...], preferred_element_type=jnp.float32) + b1[...],
        0.0).astype(jnp.bfloat16)
    out[...] = jnp.dot(h, w2[...], preferred_element_type=jnp.float32) + b2[...]


_mlp = pl.pallas_call(
    _mlp_body,
    grid=(B // BM,),
    in_specs=[pl.BlockSpec((BM, DP // 2), lambda i: (i, 0)) for _ in range(4)] + [
        pl.BlockSpec((4 * D, H), lambda i: (0, 0)),
        pl.BlockSpec((1, H), lambda i: (0, 0)),
        pl.BlockSpec((H, H), lambda i: (0, 0)),
        pl.BlockSpec((1, H), lambda i: (0, 0)),
    ],
    out_specs=pl.BlockSpec((BM, H), lambda i: (i, 0)),
    out_shape=jax.ShapeDtypeStruct((B, H), jnp.float32),
)


BV = 8192


def _pack_bits(x):
    r = x.astype(jnp.bfloat16).astype(jnp.float32)
    return jax.lax.bitcast_convert_type(r, jnp.int32)


def _relayout_body(xt, out):
    t = xt[...].T
    lo = _pack_bits(t[:, :128])
    hi = _pack_bits(jnp.concatenate(
        [t[:, 128:], jnp.zeros((t.shape[0], DP - D), jnp.float32)], axis=-1))
    word = jax.lax.shift_right_logical(lo, 16) | (hi & jnp.int32(-65536))
    out[...] = jax.lax.bitcast_convert_type(word, jnp.float32)


def _relayout(emb):
    v = emb.shape[0]
    nb = (v + BV - 1) // BV
    return pl.pallas_call(
        _relayout_body,
        grid=(nb,),
        in_specs=[pl.BlockSpec((D, BV), lambda i: (0, i))],
        out_specs=pl.BlockSpec((BV, DP // 2), lambda i: (i, 0)),
        out_shape=jax.ShapeDtypeStruct((v, DP // 2), jnp.float32),
    )(emb.T)


def kernel(category, brand, item_id, seller,
           emb_category, emb_brand, emb_item_id, emb_seller,
           W1, b1, W2, b2):
    idx = [x.astype(jnp.int32) for x in (category, brand, item_id, seller)]
    parts = []
    prev = None
    for i, e in enumerate((emb_category, emb_brand, emb_item_id, emb_seller)):
        if prev is not None:
            e, _ = jax.lax.optimization_barrier((e, prev))
        t = _relayout(e)
        prev = t
        parts.append(_gather1(idx[i], t))
    w1z = W1.astype(jnp.bfloat16)
    return _mlp(*parts, w1z, b1.reshape(1, H),
                W2.astype(jnp.bfloat16), b2.reshape(1, H))


# speedup vs baseline: 6.5230x; 1.0005x over previous
import jax
import jax.numpy as jnp
from jax import lax
from jax.experimental import pallas as pl
from jax.experimental.pallas import tpu as pltpu
from jax.experimental.pallas import tpu_sc as plsc

B = 16384
D = 192
DP = 256
H = 768
NC = 2
NS = 16
NW = NC * NS
BPW = B // NW
CH = 256


NCH = BPW // CH


def _gather_body(idx_hbm, tab, out, idx_v, buf0, buf1, sem0, sem1):
    wid = lax.axis_index("s") * NC + lax.axis_index("c")
    base = wid * BPW
    pltpu.sync_copy(idx_hbm.at[pl.ds(base, BPW)], idx_v)
    bufs, sems = (buf0, buf1), (sem0, sem1)
    copies = [None] * NCH
    for ch in range(NCH):
        ids = idx_v.at[pl.ds(ch * CH, CH)]
        copies[ch] = pltpu.async_copy(tab.at[ids], bufs[ch % 2], sems[ch % 2])
        if ch > 0:
            copies[ch - 1].wait()
            pltpu.sync_copy(bufs[(ch - 1) % 2],
                            out.at[pl.ds(base + (ch - 1) * CH, CH)])
    copies[NCH - 1].wait()
    pltpu.sync_copy(bufs[(NCH - 1) % 2],
                    out.at[pl.ds(base + (NCH - 1) * CH, CH)])


_gather1 = pl.kernel(
    _gather_body,
    mesh=plsc.VectorSubcoreMesh(core_axis_name="c", subcore_axis_name="s"),
    out_type=jax.ShapeDtypeStruct((B, DP // 2), jnp.float32),
    scratch_types=[
        pltpu.VMEM((BPW,), jnp.int32),
        pltpu.VMEM((CH, DP // 2), jnp.float32),
        pltpu.VMEM((CH, DP // 2), jnp.float32),
        pltpu.SemaphoreType.DMA,
        pltpu.SemaphoreType.DMA,
    ],
)


BM = 2048


def _unpack2(p):
    w = jax.lax.bitcast_convert_type(p[...], jnp.int32)
    lo = jax.lax.bitcast_convert_type(
        jax.lax.shift_left(w, 16), jnp.float32)
    hi = jax.lax.bitcast_convert_type(w & jnp.int32(-65536), jnp.float32)
    return lo, hi


def _mlp_body(xc, xb, xi, xs, w1, b1, w2, b2, out):
    pieces = []
    for p in (xc, xb, xi, xs):
        lo, hi = _unpack2(p)
        pieces.append(lo)
        pieces.append(hi[:, :D - 128])
    x = jnp.concatenate(pieces, axis=-1).astype(jnp.bfloat16)
    h = jnp.maximum(
        jnp.dot(x, w1[...], preferred_element_type=jnp.float32) + b1[...],
        0.0).astype(jnp.bfloat16)
    out[...] = jnp.dot(h, w2[...], preferred_element_type=jnp.float32) + b2[...]


_mlp = pl.pallas_call(
    _mlp_body,
    grid=(B // BM,),
    in_specs=[pl.BlockSpec((BM, DP // 2), lambda i: (i, 0)) for _ in range(4)] + [
        pl.BlockSpec((4 * D, H), lambda i: (0, 0)),
        pl.BlockSpec((1, H), lambda i: (0, 0)),
        pl.BlockSpec((H, H), lambda i: (0, 0)),
        pl.BlockSpec((1, H), lambda i: (0, 0)),
    ],
    out_specs=pl.BlockSpec((BM, H), lambda i: (i, 0)),
    out_shape=jax.ShapeDtypeStruct((B, H), jnp.float32),
)


BV = 8192


def _pack_bits(x):
    r = x.astype(jnp.bfloat16).astype(jnp.float32)
    return jax.lax.bitcast_convert_type(r, jnp.int32)


def _relayout_body(xt, out):
    t = xt[...].T
    lo = _pack_bits(t[:, :128])
    hi = _pack_bits(jnp.concatenate(
        [t[:, 128:], jnp.zeros((t.shape[0], DP - D), jnp.float32)], axis=-1))
    word = jax.lax.shift_right_logical(lo, 16) | (hi & jnp.int32(-65536))
    out[...] = jax.lax.bitcast_convert_type(word, jnp.float32)


def _relayout(emb):
    v = emb.shape[0]
    nb = (v + BV - 1) // BV
    return pl.pallas_call(
        _relayout_body,
        grid=(nb,),
        in_specs=[pl.BlockSpec((D, BV), lambda i: (0, i))],
        out_specs=pl.BlockSpec((BV, DP // 2), lambda i: (i, 0)),
        out_shape=jax.ShapeDtypeStruct((v, DP // 2), jnp.float32),
    )(emb.T)


def kernel(category, brand, item_id, seller,
           emb_category, emb_brand, emb_item_id, emb_seller,
           W1, b1, W2, b2):
    idx = [x.astype(jnp.int32) for x in (category, brand, item_id, seller)]
    parts = []
    prev = None
    for i, e in enumerate((emb_category, emb_brand, emb_item_id, emb_seller)):
        if prev is not None:
            e, _ = jax.lax.optimization_barrier((e, prev))
        t = _relayout(e)
        prev = t
        parts.append(_gather1(idx[i], t))
    return _mlp(*parts, W1.astype(jnp.bfloat16), b1.reshape(1, H),
                W2.astype(jnp.bfloat16), b2.reshape(1, H))
